# baseline (speedup 1.0000x reference)
import jax
import jax.numpy as jnp
from jax import lax
from jax.experimental import pallas as pl
from jax.experimental.pallas import tpu as pltpu
from jax.experimental.pallas import tpu_sc as plsc

_B = 16384
_D = 64
_NC = 2
_NS = 16
_NW = _NC * _NS
_BPW = _B // _NW
_CHUNK = 128
_NCHUNK = _BPW // _CHUNK
_LANES = 16
_SCALE = 0.5 * 0.5 / _B


def _sc_body(x_hbm, idx_hbm, off_hbm, cent_hbm, out_hbm,
             idx_v, off_v, c_v, x_v, acc_v, sem):
    wid = lax.axis_index("s") * _NC + lax.axis_index("c")
    base = wid * _BPW

    pltpu.sync_copy(idx_hbm.at[wid], idx_v)
    copies = [
        pltpu.async_copy(
            cent_hbm.at[idx_v.at[j]], c_v.at[pl.ds(j * _CHUNK, _CHUNK)], sem
        )
        for j in range(_NCHUNK)
    ]
    pltpu.sync_copy(off_hbm.at[pl.ds(base, _BPW)], off_v)
    pltpu.sync_copy(x_hbm.at[pl.ds(base * _D, _BPW * _D)], x_v)
    for c in copies:
        c.wait()

    def group(g, acc):
        off16 = off_v[pl.ds(g * _LANES, _LANES)]
        for k in range(_LANES):
            r = g * _LANES + k
            off = off16[k]
            for j in range(_D // _LANES):
                d = (x_v[pl.ds(r * _D + j * _LANES, _LANES)]
                     - c_v[r, pl.ds(off + j * _LANES, _LANES)])
                acc = acc + d * d
        return acc

    acc = lax.fori_loop(0, _BPW // _LANES, group,
                        jnp.zeros((_LANES,), jnp.float32))
    acc_v[...] = acc
    pltpu.sync_copy(acc_v, out_hbm.at[wid])


@jax.jit
def _center_loss(x, labels_i32, centers):
    mesh = plsc.VectorSubcoreMesh(core_axis_name="c", subcore_axis_name="s")
    idx = (labels_i32 >> 1).reshape(_NW, _NCHUNK, _CHUNK)
    off = (labels_i32 & 1) * _D
    cent2 = centers.reshape(centers.shape[0] // 2, 2 * _D)
    partials = pl.kernel(
        _sc_body,
        out_type=jax.ShapeDtypeStruct((_NW, _LANES), jnp.float32),
        mesh=mesh,
        scratch_types=[
            pltpu.VMEM((_NCHUNK, _CHUNK), jnp.int32),
            pltpu.VMEM((_BPW,), jnp.int32),
            pltpu.VMEM((_BPW, 2 * _D), jnp.float32),
            pltpu.VMEM((_BPW * _D,), jnp.float32),
            pltpu.VMEM((_LANES,), jnp.float32),
            pltpu.SemaphoreType.DMA,
        ],
    )(x.reshape(-1), idx, off, cent2)
    return _SCALE * jnp.sum(partials)


def kernel(x, labels, centers):
    return _center_loss(x, labels.astype(jnp.int32), centers)


# speedup vs baseline: 1.0081x; 1.0081x over previous
import jax
import jax.numpy as jnp
from jax import lax
from jax.experimental import pallas as pl
from jax.experimental.pallas import tpu as pltpu
from jax.experimental.pallas import tpu_sc as plsc

_B = 16384
_D = 64
_NC = 2
_NS = 16
_NW = _NC * _NS
_BPW = _B // _NW
_CHUNK = 128
_NCHUNK = _BPW // _CHUNK
_LANES = 16
_SCALE = 0.5 * 0.5 / _B


def _sc_body(x_hbm, idx_hbm, cent_hbm, out_hbm, idx_v, c_v, x_v, acc_v, sem):
    wid = lax.axis_index("s") * _NC + lax.axis_index("c")
    base = wid * _BPW

    pltpu.sync_copy(idx_hbm.at[wid], idx_v)
    copies = [
        pltpu.async_copy(
            cent_hbm.at[idx_v.at[j]], c_v.at[pl.ds(j * _CHUNK, _CHUNK)], sem
        )
        for j in range(_NCHUNK)
    ]
    pltpu.sync_copy(x_hbm.at[pl.ds(base * _D, _BPW * _D)], x_v)
    for c in copies:
        c.wait()

    def row(r, acc):
        for j in range(_D // _LANES):
            d = (x_v[pl.ds(r * _D + j * _LANES, _LANES)]
                 - c_v[r, pl.ds(j * _LANES, _LANES)])
            acc = acc + d * d
        return acc

    acc = lax.fori_loop(0, _BPW, row, jnp.zeros((_LANES,), jnp.float32))
    acc_v[...] = acc
    pltpu.sync_copy(acc_v, out_hbm.at[wid])


@jax.jit
def _center_loss(x, labels_i32, centers):
    mesh = plsc.VectorSubcoreMesh(core_axis_name="c", subcore_axis_name="s")
    idx = labels_i32.reshape(_NW, _NCHUNK, _CHUNK)
    partials = pl.kernel(
        _sc_body,
        out_type=jax.ShapeDtypeStruct((_NW, _LANES), jnp.float32),
        mesh=mesh,
        compiler_params=pltpu.CompilerParams(use_tc_tiling_on_sc=False),
        scratch_types=[
            pltpu.VMEM((_NCHUNK, _CHUNK), jnp.int32),
            pltpu.VMEM((_BPW, _D), jnp.float32),
            pltpu.VMEM((_BPW * _D,), jnp.float32),
            pltpu.VMEM((_LANES,), jnp.float32),
            pltpu.SemaphoreType.DMA,
        ],
    )(x.reshape(-1), idx, centers)
    return _SCALE * jnp.sum(partials)


def kernel(x, labels, centers):
    return _center_loss(x, labels.astype(jnp.int32), centers)
